# baseline (speedup 1.0000x reference)
import functools
import math

import jax
import jax.numpy as jnp
from jax import lax
from jax.experimental import pallas as pl
from jax.experimental.pallas import tpu as pltpu
from jax.experimental.pallas import tpu_sc as plsc

N = 10000
D = 128
DEG_MAX = 64
OMEGA = 0.01
K_FRAC = 0.5
B_IDX = 2500
BM = 400
NM = N // BM
SQRT_M = math.sqrt(128.0)
NW = 32
PER_W = 80
L = 16


def _main_body(adj_ref, x_ref, deg_ref, w_ref, b_ref, pe_ref,
               wg_ref, bg_ref, wb_ref, bb_ref, wa_ref, wr_ref,
               out_ref, nrm_ref, ft_ref,
               h_s, gt_s, bt_s, ft_s, kthr_s):
    m = pl.program_id(0)

    @pl.when(m == 0)
    def _prologue():
        h = jnp.dot(x_ref[...], w_ref[...], preferred_element_type=jnp.float32)
        h_s[...] = (h + b_ref[...]) * SQRT_M
        g = jnp.dot(pe_ref[...], wg_ref[...], preferred_element_type=jnp.float32) + bg_ref[...]
        g = jnp.where(g >= 0.0, g, 0.01 * g)
        bt = jnp.dot(pe_ref[...], wb_ref[...], preferred_element_type=jnp.float32) + bb_ref[...]
        bt = jnp.where(bt >= 0.0, bt, 0.01 * bt)
        gt_s[...] = g
        bt_s[...] = bt
        ft = (jnp.sqrt(jnp.sum(g * g, axis=1, keepdims=True))
              + jnp.sqrt(jnp.sum(bt * bt, axis=1, keepdims=True)))
        ft_s[...] = ft
        ft_ref[...] = ft
        kthr_s[...] = (jnp.sum(deg_ref[...].astype(jnp.float32), keepdims=True)
                       .reshape(1, 1) * (K_FRAC / N))

    agg = jnp.dot(adj_ref[...], h_s[...], preferred_element_type=jnp.float32)
    degi = deg_ref[pl.ds(m * BM, BM), :]
    deg = degi.astype(jnp.float32)
    hm = h_s[pl.ds(m * BM, BM), :]
    inv = jnp.where(deg > 0.0, 1.0 / deg, 0.0)
    iv = agg * inv
    io = jax.lax.broadcasted_iota(jnp.int32, (BM, DEG_MAX), 1)
    oh = (degi == io).astype(jnp.float32)
    gamma = jnp.dot(oh, gt_s[...], preferred_element_type=jnp.float32)
    beta = jnp.dot(oh, bt_s[...], preferred_element_type=jnp.float32)
    g1 = gamma + 1.0
    ba = g1 * jnp.dot(iv, wa_ref[...], preferred_element_type=jnp.float32) + beta
    br = g1 * jnp.dot(iv, wr_ref[...], preferred_element_type=jnp.float32) + beta
    r = (deg < kthr_s[0, 0]).astype(jnp.float32)
    bias = OMEGA * (r * ba - (1.0 - r) * br)
    out_ref[...] = (agg + hm + bias) / (deg + 1.0)
    na = jnp.sqrt(jnp.sum(ba * ba, axis=1, keepdims=True))
    nr = jnp.sqrt(jnp.sum(br * br, axis=1, keepdims=True))
    nrm_ref[pl.ds(m, 1), :] = (r * na + (1.0 - r) * nr).reshape(1, BM)


def _sc_loss_body(nrm_hbm, deg_hbm, ft_hbm, idx_hbm, outb_hbm, outf_hbm,
                  nrm_v, deg_v, ft_v, idx_v, accb_v, accf_v):
    c = lax.axis_index("c")
    s = lax.axis_index("s")
    wid = s * 2 + c
    base = wid * PER_W
    pltpu.sync_copy(nrm_hbm, nrm_v)
    pltpu.sync_copy(deg_hbm, deg_v)
    pltpu.sync_copy(ft_hbm, ft_v)
    pltpu.sync_copy(idx_hbm.at[pl.ds(base, PER_W)], idx_v)
    accb = jnp.zeros((L,), jnp.float32)
    accf = jnp.zeros((L,), jnp.float32)
    for k in range(PER_W // L):
        idxc = idx_v[pl.ds(k * L, L)]
        ok = idxc >= 0
        idxs = jnp.where(ok, idxc, 0)
        vals = plsc.load_gather(nrm_v, [idxs])
        dc = plsc.load_gather(deg_v, [idxs])
        fv = plsc.load_gather(ft_v, [dc])
        accb = accb + jnp.where(ok, vals, 0.0)
        accf = accf + jnp.where(ok, fv, 0.0)
    accb_v[...] = accb
    accf_v[...] = accf
    pltpu.sync_copy(accb_v, outb_hbm.at[pl.ds(wid * L, L)])
    pltpu.sync_copy(accf_v, outf_hbm.at[pl.ds(wid * L, L)])


_sc_loss = functools.partial(
    pl.kernel,
    mesh=plsc.VectorSubcoreMesh(core_axis_name="c", subcore_axis_name="s"),
    out_type=[
        jax.ShapeDtypeStruct((NW * L,), jnp.float32),
        jax.ShapeDtypeStruct((NW * L,), jnp.float32),
    ],
    scratch_types=[
        pltpu.VMEM((N,), jnp.float32),
        pltpu.VMEM((N,), jnp.int32),
        pltpu.VMEM((DEG_MAX,), jnp.float32),
        pltpu.VMEM((PER_W,), jnp.int32),
        pltpu.VMEM((L,), jnp.float32),
        pltpu.VMEM((L,), jnp.float32),
    ],
    compiler_params=pltpu.CompilerParams(needs_layout_passes=False),
)(_sc_loss_body)


def kernel(x, adj, degree, idx, edge, W, b, W_gamma, W_beta, b_gamma, b_beta,
           W_add, W_rev, PE):
    f32 = jnp.float32
    pe64 = PE[:DEG_MAX]
    b2 = b.reshape(1, D)
    degi = degree.astype(jnp.int32)

    out, nrm25, ft = pl.pallas_call(
        _main_body,
        grid=(NM,),
        in_specs=[
            pl.BlockSpec((BM, N), lambda m: (m, 0)),
            pl.BlockSpec((N, D), lambda m: (0, 0)),
            pl.BlockSpec((N, 1), lambda m: (0, 0)),
            pl.BlockSpec((D, D), lambda m: (0, 0)),
            pl.BlockSpec((1, D), lambda m: (0, 0)),
            pl.BlockSpec((DEG_MAX, D), lambda m: (0, 0)),
            pl.BlockSpec((D, D), lambda m: (0, 0)),
            pl.BlockSpec((1, D), lambda m: (0, 0)),
            pl.BlockSpec((D, D), lambda m: (0, 0)),
            pl.BlockSpec((1, D), lambda m: (0, 0)),
            pl.BlockSpec((D, D), lambda m: (0, 0)),
            pl.BlockSpec((D, D), lambda m: (0, 0)),
        ],
        out_specs=[
            pl.BlockSpec((BM, D), lambda m: (m, 0)),
            pl.BlockSpec((NM, BM), lambda m: (0, 0)),
            pl.BlockSpec((DEG_MAX, 1), lambda m: (0, 0)),
        ],
        out_shape=[
            jax.ShapeDtypeStruct((N, D), f32),
            jax.ShapeDtypeStruct((NM, BM), f32),
            jax.ShapeDtypeStruct((DEG_MAX, 1), f32),
        ],
        scratch_shapes=[
            pltpu.VMEM((N, D), f32),
            pltpu.VMEM((DEG_MAX, D), f32),
            pltpu.VMEM((DEG_MAX, D), f32),
            pltpu.VMEM((DEG_MAX, 1), f32),
            pltpu.VMEM((1, 1), f32),
        ],
        compiler_params=pltpu.CompilerParams(
            dimension_semantics=("arbitrary",),
        ),
    )(adj, x, degi, W, b2, pe64, W_gamma, b_gamma, W_beta, b_beta,
      W_add, W_rev)

    idx_pad = jnp.concatenate(
        [idx.astype(jnp.int32),
         jnp.full((NW * PER_W - B_IDX,), -1, jnp.int32)])
    pb, pf = _sc_loss(nrm25.reshape(N), degi.reshape(N), ft.reshape(DEG_MAX),
                      idx_pad)
    return out, jnp.sum(pb) * (1.0 / B_IDX), jnp.sum(pf) * (1.0 / B_IDX)


# speedup vs baseline: 1.1879x; 1.1879x over previous
import math

import jax
import jax.numpy as jnp
from jax.experimental import pallas as pl
from jax.experimental.pallas import tpu as pltpu

N = 10000
D = 128
DEG_MAX = 64
OMEGA = 0.01
K_FRAC = 0.5
B_IDX = 2500
BM = 400
NM = N // BM
SQRT_M = math.sqrt(128.0)


def _main_body(adj_ref, x_ref, deg_ref, idx_ref, w_ref, b_ref, pe_ref,
               wg_ref, bg_ref, wb_ref, bb_ref, wa_ref, wr_ref,
               out_ref, lbf_ref,
               h_s, gt_s, bt_s, ft_s, kthr_s, nrm_s, film_s):
    m = pl.program_id(0)

    @pl.when(m == 0)
    def _prologue():
        h = jnp.dot(x_ref[...], w_ref[...], preferred_element_type=jnp.float32)
        h_s[...] = (h + b_ref[...]) * SQRT_M
        g = jnp.dot(pe_ref[...], wg_ref[...], preferred_element_type=jnp.float32) + bg_ref[...]
        g = jnp.where(g >= 0.0, g, 0.01 * g)
        bt = jnp.dot(pe_ref[...], wb_ref[...], preferred_element_type=jnp.float32) + bb_ref[...]
        bt = jnp.where(bt >= 0.0, bt, 0.01 * bt)
        gt_s[...] = g
        bt_s[...] = bt
        ft_s[...] = (jnp.sqrt(jnp.sum(g * g, axis=1, keepdims=True))
                     + jnp.sqrt(jnp.sum(bt * bt, axis=1, keepdims=True)))
        kthr_s[...] = (jnp.sum(deg_ref[...].astype(jnp.float32), keepdims=True)
                       .reshape(1, 1) * (K_FRAC / N))

    agg = jnp.dot(adj_ref[...], h_s[...], preferred_element_type=jnp.float32)
    degi = deg_ref[pl.ds(m * BM, BM), :]
    deg = degi.astype(jnp.float32)
    hm = h_s[pl.ds(m * BM, BM), :]
    inv = jnp.where(deg > 0.0, 1.0 / deg, 0.0)
    iv = agg * inv
    io = jax.lax.broadcasted_iota(jnp.int32, (BM, DEG_MAX), 1)
    oh = (degi == io).astype(jnp.float32)
    gamma = jnp.dot(oh, gt_s[...], preferred_element_type=jnp.float32)
    beta = jnp.dot(oh, bt_s[...], preferred_element_type=jnp.float32)
    g1 = gamma + 1.0
    ba = g1 * jnp.dot(iv, wa_ref[...], preferred_element_type=jnp.float32) + beta
    br = g1 * jnp.dot(iv, wr_ref[...], preferred_element_type=jnp.float32) + beta
    r = (deg < kthr_s[0, 0]).astype(jnp.float32)
    bias = OMEGA * (r * ba - (1.0 - r) * br)
    out_ref[...] = (agg + hm + bias) / (deg + 1.0)
    na = jnp.sqrt(jnp.sum(ba * ba, axis=1, keepdims=True))
    nr = jnp.sqrt(jnp.sum(br * br, axis=1, keepdims=True))
    nrm_s[pl.ds(m, 1), :] = (r * na + (1.0 - r) * nr).reshape(1, BM)
    film_s[pl.ds(m, 1), :] = jnp.dot(
        oh, ft_s[...], preferred_element_type=jnp.float32).reshape(1, BM)

    @pl.when(m == NM - 1)
    def _loss():
        idx = idx_ref[...]
        hi = idx // BM
        lo = idx - hi * BM
        ioh = jax.lax.broadcasted_iota(jnp.int32, (B_IDX, NM), 1)
        iol = jax.lax.broadcasted_iota(jnp.int32, (B_IDX, BM), 1)
        oh_hi = (hi == ioh).astype(jnp.float32)
        oh_lo = (lo == iol).astype(jnp.float32)
        tb = jnp.dot(oh_hi, nrm_s[...], preferred_element_type=jnp.float32)
        tf = jnp.dot(oh_hi, film_s[...], preferred_element_type=jnp.float32)
        lb = jnp.sum(tb * oh_lo, keepdims=True).reshape(1, 1)
        lf = jnp.sum(tf * oh_lo, keepdims=True).reshape(1, 1)
        lbf_ref[...] = jnp.concatenate([lb, lf], axis=1) * (1.0 / B_IDX)


def kernel(x, adj, degree, idx, edge, W, b, W_gamma, W_beta, b_gamma, b_beta,
           W_add, W_rev, PE):
    f32 = jnp.float32
    pe64 = PE[:DEG_MAX]
    b2 = b.reshape(1, D)
    degi = degree.astype(jnp.int32)
    idx2 = idx.reshape(B_IDX, 1).astype(jnp.int32)

    out, lbf = pl.pallas_call(
        _main_body,
        grid=(NM,),
        in_specs=[
            pl.BlockSpec((BM, N), lambda m: (m, 0)),
            pl.BlockSpec((N, D), lambda m: (0, 0)),
            pl.BlockSpec((N, 1), lambda m: (0, 0)),
            pl.BlockSpec((B_IDX, 1), lambda m: (0, 0)),
            pl.BlockSpec((D, D), lambda m: (0, 0)),
            pl.BlockSpec((1, D), lambda m: (0, 0)),
            pl.BlockSpec((DEG_MAX, D), lambda m: (0, 0)),
            pl.BlockSpec((D, D), lambda m: (0, 0)),
            pl.BlockSpec((1, D), lambda m: (0, 0)),
            pl.BlockSpec((D, D), lambda m: (0, 0)),
            pl.BlockSpec((1, D), lambda m: (0, 0)),
            pl.BlockSpec((D, D), lambda m: (0, 0)),
            pl.BlockSpec((D, D), lambda m: (0, 0)),
        ],
        out_specs=[
            pl.BlockSpec((BM, D), lambda m: (m, 0)),
            pl.BlockSpec((1, 2), lambda m: (0, 0)),
        ],
        out_shape=[
            jax.ShapeDtypeStruct((N, D), f32),
            jax.ShapeDtypeStruct((1, 2), f32),
        ],
        scratch_shapes=[
            pltpu.VMEM((N, D), f32),
            pltpu.VMEM((DEG_MAX, D), f32),
            pltpu.VMEM((DEG_MAX, D), f32),
            pltpu.VMEM((DEG_MAX, 1), f32),
            pltpu.VMEM((1, 1), f32),
            pltpu.VMEM((NM, BM), f32),
            pltpu.VMEM((NM, BM), f32),
        ],
        compiler_params=pltpu.CompilerParams(
            dimension_semantics=("arbitrary",),
        ),
    )(adj, x, degi, idx2, W, b2, pe64, W_gamma, b_gamma, W_beta, b_beta,
      W_add, W_rev)

    return out, lbf[0, 0], lbf[0, 1]
